# baseline (speedup 1.0000x reference)
import functools

import jax
import jax.numpy as jnp
from jax import lax
from jax.experimental import pallas as pl
from jax.experimental.pallas import tpu as pltpu
from jax.experimental.pallas import tpu_sc as plsc

B = 16384
NCAT = 26
VOCAB = 100000
EMB = 16
NUM_DIM = 13
TAB_IN = NUM_DIM + NCAT * EMB
H1 = 256
H2 = 256
OUT = 128
EPS = 1e-5

NPLANES = NCAT * EMB
RB = NPLANES // 8
NC = 2
NS = 16
NW = NC * NS
PPT = NPLANES // NW
CH = 2048
NCH = B // CH
L = 16
U = 8
VH0 = 49920
VH1 = VOCAB - VH0
HMAX = VH1


def _sc_gather(table3, idx26):
    mesh = plsc.VectorSubcoreMesh(core_axis_name="c", subcore_axis_name="s")

    @functools.partial(
        pl.kernel,
        out_type=jax.ShapeDtypeStruct((NPLANES, B), jnp.float32),
        mesh=mesh,
        scratch_types=[
            pltpu.VMEM((HMAX,), jnp.float32),
            pltpu.VMEM((HMAX,), jnp.float32),
            pltpu.VMEM((2, CH), jnp.int32),
            pltpu.VMEM((B,), jnp.float32),
            pltpu.SemaphoreType.DMA,
            pltpu.SemaphoreType.DMA,
            pltpu.SemaphoreType.DMA,
        ],
        compiler_params=pltpu.CompilerParams(
            needs_layout_passes=False,
            disable_bounds_checks=True,
        ),
    )
    def k(table_hbm, idx_hbm, out_hbm, pv0, pv1, idx_v, out_v, semp, semi, semw):
        wid = lax.axis_index("s") * NC + lax.axis_index("c")

        def half_copy(plane, h):
            off = h * VH0
            ln = VH1 if h else VH0
            pv = pv1 if h else pv0
            return (table_hbm.at[plane // 8, plane % 8, pl.ds(off, ln)],
                    pv.at[pl.ds(0, ln)])

        def half_start(plane, h):
            src, dst = half_copy(plane, h)
            pltpu.async_copy(src, dst, semp)

        def half_wait(plane, h):
            src, dst = half_copy(plane, h)
            pltpu.make_async_copy(src, dst, semp).wait()

        def idx_start(f, c):
            pltpu.async_copy(
                idx_hbm.at[f, pl.ds(c * CH, CH)], idx_v.at[c % 2], semi)

        def idx_wait(f, c):
            pltpu.make_async_copy(
                idx_hbm.at[f, pl.ds(c * CH, CH)], idx_v.at[c % 2], semi).wait()

        def out_wait(plane):
            pltpu.make_async_copy(out_v, out_hbm.at[plane], semw).wait()

        def gather_half(f, h):
            def per_chunk(c, carry2):
                idx_wait(f, c)

                @pl.when(c + 1 < NCH)
                def _():
                    idx_start(f, c + 1)

                buf = c % 2

                def per_vec(q, carry3):
                    for u in range(U):
                        o = q * (L * U) + u * L
                        iv = idx_v[buf, pl.ds(o, L)]
                        if h == 0:
                            msk = iv < VH0
                            vals = plsc.load_gather(pv0, [iv],
                                                    mask=msk)
                            out_v[pl.ds(c * CH + o, L)] = jnp.where(
                                msk, vals, 0.0)
                        else:
                            msk = iv >= VH0
                            vals = plsc.load_gather(pv1, [iv - VH0],
                                                    mask=msk)
                            prev = out_v[pl.ds(c * CH + o, L)]
                            out_v[pl.ds(c * CH + o, L)] = prev + jnp.where(
                                msk, vals, 0.0)
                    return carry3

                lax.fori_loop(0, CH // (L * U), per_vec, 0)
                return carry2

            lax.fori_loop(0, NCH, per_chunk, 0)

        half_start(wid * PPT, 0)

        def per_plane(j, carry):
            plane = wid * PPT + j
            f = plane // EMB
            idx_start(f, 0)
            half_wait(plane, 0)
            half_start(plane, 1)

            @pl.when(j > 0)
            def _():
                out_wait(plane - 1)

            gather_half(f, 0)
            idx_start(f, 0)
            half_wait(plane, 1)

            @pl.when(j + 1 < PPT)
            def _():
                half_start(plane + 1, 0)

            gather_half(f, 1)
            pltpu.async_copy(out_v, out_hbm.at[plane], semw)
            return carry

        lax.fori_loop(0, PPT, per_plane, 0)
        out_wait(wid * PPT + PPT - 1)

    return k(table3, idx26)


def _mlp_body(nxt_ref, ct_ref, g0n_r, b0n_r, g0c_r, b0c_r, w1n_r, w1c_r, b1_r,
              g1_r, bb1_r, w2_r, b2_r, g2_r, bb2_r, w3_r, b3_r, out_ref):
    f32 = jnp.float32
    pr = lax.Precision.DEFAULT
    dnt = (((0,), (0,)), ((), ()))
    nxt = nxt_ref[...]
    ctt = ct_ref[...]
    s = (jnp.sum(nxt, axis=0, keepdims=True)
         + jnp.sum(ctt, axis=0, keepdims=True))
    ss = (jnp.sum(nxt * nxt, axis=0, keepdims=True)
          + jnp.sum(ctt * ctt, axis=0, keepdims=True))
    m = s * (1.0 / TAB_IN)
    v = ss * (1.0 / TAB_IN) - m * m
    inv = lax.rsqrt(v + EPS)
    ynt = (nxt - m) * inv * g0n_r[...] + b0n_r[...]
    yct = (ctt - m) * inv * g0c_r[...] + b0c_r[...]
    h = (lax.dot_general(ynt, w1n_r[...], dnt, precision=pr,
                         preferred_element_type=f32)
         + lax.dot_general(yct, w1c_r[...], dnt, precision=pr,
                           preferred_element_type=f32)
         + b1_r[...])
    h = h * jax.nn.sigmoid(h)
    m1 = jnp.mean(h, axis=1, keepdims=True)
    v1 = jnp.mean(h * h, axis=1, keepdims=True) - m1 * m1
    h = (h - m1) * lax.rsqrt(v1 + EPS) * g1_r[...] + bb1_r[...]
    h = jnp.dot(h, w2_r[...], preferred_element_type=f32, precision=pr) + b2_r[...]
    h = h * jax.nn.sigmoid(h)
    m2 = jnp.mean(h, axis=1, keepdims=True)
    v2 = jnp.mean(h * h, axis=1, keepdims=True) - m2 * m2
    h = (h - m2) * lax.rsqrt(v2 + EPS) * g2_r[...] + bb2_r[...]
    out_ref[...] = jnp.dot(h, w3_r[...], preferred_element_type=f32, precision=pr) + b3_r[...]


def _mlp(nxt, catT, g0n, b0n, g0c, b0c, W1n, W1c, b1, g1, bb1, W2, b2, g2, bb2, W3, b3):
    R = 512
    grid = (B // R,)
    col_blk = lambda shape: pl.BlockSpec(shape, lambda i: (0, i))
    full = lambda shape: pl.BlockSpec(shape, lambda i: (0, 0))
    return pl.pallas_call(
        _mlp_body,
        grid=grid,
        in_specs=[
            col_blk((NUM_DIM, R)),
            col_blk((NPLANES, R)),
            full((NUM_DIM, 1)), full((NUM_DIM, 1)),
            full((NPLANES, 1)), full((NPLANES, 1)),
            full((NUM_DIM, H1)), full((NPLANES, H1)), full((1, H1)),
            full((1, H1)), full((1, H1)),
            full((H1, H2)), full((1, H2)),
            full((1, H2)), full((1, H2)),
            full((H2, OUT)), full((1, OUT)),
        ],
        out_specs=pl.BlockSpec((R, OUT), lambda i: (i, 0)),
        out_shape=jax.ShapeDtypeStruct((B, OUT), jnp.float32),
        compiler_params=pltpu.CompilerParams(
            dimension_semantics=("arbitrary",),
        ),
    )(nxt, catT, g0n, b0n, g0c, b0c, W1n, W1c, b1, g1, bb1, W2, b2, g2, bb2, W3, b3)


def kernel(numeric_tensor, categorical_idx, tables, ln0_g, ln0_b, W1, b1,
           ln1_g, ln1_b, W2, b2, ln2_g, ln2_b, W3, b3):
    i32 = jnp.int32
    table3 = tables.transpose(0, 2, 1).reshape(RB, 8, VOCAB)
    idx26 = categorical_idx.astype(i32)

    catT = _sc_gather(table3, idx26)

    c1 = lambda a: a.reshape(-1, 1)
    r1 = lambda a: a.reshape(1, -1)
    return _mlp(
        numeric_tensor.T, catT,
        c1(ln0_g[:NUM_DIM]), c1(ln0_b[:NUM_DIM]),
        c1(ln0_g[NUM_DIM:]), c1(ln0_b[NUM_DIM:]),
        W1[:NUM_DIM], W1[NUM_DIM:], r1(b1),
        r1(ln1_g), r1(ln1_b),
        W2, r1(b2),
        r1(ln2_g), r1(ln2_b),
        W3, r1(b3),
    )


# speedup vs baseline: 1.4747x; 1.4747x over previous
import functools

import jax
import jax.numpy as jnp
from jax import lax
from jax.experimental import pallas as pl
from jax.experimental.pallas import tpu as pltpu
from jax.experimental.pallas import tpu_sc as plsc

B = 16384
NCAT = 26
VOCAB = 100000
EMB = 16
NUM_DIM = 13
TAB_IN = NUM_DIM + NCAT * EMB
H1 = 256
H2 = 256
OUT = 128
EPS = 1e-5

NPLANES = NCAT * EMB
RB = NPLANES // 8
NC = 2
NS = 16
NW = NC * NS
PPT = NPLANES // NW
CH = 4096
NCH = B // CH
L = 16
U = 8


def _sc_gather(table3, idx26):
    mesh = plsc.VectorSubcoreMesh(core_axis_name="c", subcore_axis_name="s")

    @functools.partial(
        pl.kernel,
        out_type=jax.ShapeDtypeStruct((NPLANES, B), jnp.float32),
        mesh=mesh,
        scratch_types=[
            pltpu.VMEM((VOCAB,), jnp.float32),
            pltpu.VMEM((2, CH), jnp.int32),
            pltpu.VMEM((B,), jnp.float32),
            pltpu.SemaphoreType.DMA,
            pltpu.SemaphoreType.DMA,
        ],
        compiler_params=pltpu.CompilerParams(
            needs_layout_passes=False,
            disable_bounds_checks=True,
        ),
    )
    def k(table_hbm, idx_hbm, out_hbm, plane_v, idx_v, out_v, semi, semw):
        wid = lax.axis_index("s") * NC + lax.axis_index("c")

        def idx_start(f, c):
            pltpu.async_copy(
                idx_hbm.at[f, pl.ds(c * CH, CH)], idx_v.at[c % 2], semi)

        def idx_wait(f, c):
            pltpu.make_async_copy(
                idx_hbm.at[f, pl.ds(c * CH, CH)], idx_v.at[c % 2], semi).wait()

        def out_wait(plane):
            pltpu.make_async_copy(out_v, out_hbm.at[plane], semw).wait()

        def per_plane(j, carry):
            plane = wid * PPT + j
            rb = plane // 8
            sub = plane % 8
            f = plane // EMB
            pltpu.sync_copy(table_hbm.at[rb, sub], plane_v)
            idx_start(f, 0)

            @pl.when(j > 0)
            def _():
                out_wait(plane - 1)

            def per_chunk(c, carry2):
                idx_wait(f, c)

                @pl.when(c + 1 < NCH)
                def _():
                    idx_start(f, c + 1)

                buf = c % 2

                def per_vec(q, carry3):
                    for u in range(U):
                        o = q * (L * U) + u * L
                        iv = idx_v[buf, pl.ds(o, L)]
                        vals = plsc.load_gather(plane_v, [iv])
                        out_v[pl.ds(c * CH + o, L)] = vals
                    return carry3

                lax.fori_loop(0, CH // (L * U), per_vec, 0)
                return carry2

            lax.fori_loop(0, NCH, per_chunk, 0)
            pltpu.async_copy(out_v, out_hbm.at[plane], semw)
            return carry

        lax.fori_loop(0, PPT, per_plane, 0)
        out_wait(wid * PPT + PPT - 1)

    return k(table3, idx26)


def _mlp_body(nxt_ref, ct_ref, g0n_r, b0n_r, g0c_r, b0c_r, w1n_r, w1c_r, b1_r,
              g1_r, bb1_r, w2_r, b2_r, g2_r, bb2_r, w3_r, b3_r, out_ref):
    f32 = jnp.float32
    pr = lax.Precision.DEFAULT
    dnt = (((0,), (0,)), ((), ()))
    nxt = nxt_ref[...]
    ctt = ct_ref[...]
    s = (jnp.sum(nxt, axis=0, keepdims=True)
         + jnp.sum(ctt, axis=0, keepdims=True))
    ss = (jnp.sum(nxt * nxt, axis=0, keepdims=True)
          + jnp.sum(ctt * ctt, axis=0, keepdims=True))
    m = s * (1.0 / TAB_IN)
    v = ss * (1.0 / TAB_IN) - m * m
    inv = lax.rsqrt(v + EPS)
    ynt = (nxt - m) * inv * g0n_r[...] + b0n_r[...]
    yct = (ctt - m) * inv * g0c_r[...] + b0c_r[...]
    h = (lax.dot_general(ynt, w1n_r[...], dnt, precision=pr,
                         preferred_element_type=f32)
         + lax.dot_general(yct, w1c_r[...], dnt, precision=pr,
                           preferred_element_type=f32)
         + b1_r[...])
    h = h * jax.nn.sigmoid(h)
    m1 = jnp.mean(h, axis=1, keepdims=True)
    v1 = jnp.mean(h * h, axis=1, keepdims=True) - m1 * m1
    h = (h - m1) * lax.rsqrt(v1 + EPS) * g1_r[...] + bb1_r[...]
    h = jnp.dot(h, w2_r[...], preferred_element_type=f32, precision=pr) + b2_r[...]
    h = h * jax.nn.sigmoid(h)
    m2 = jnp.mean(h, axis=1, keepdims=True)
    v2 = jnp.mean(h * h, axis=1, keepdims=True) - m2 * m2
    h = (h - m2) * lax.rsqrt(v2 + EPS) * g2_r[...] + bb2_r[...]
    out_ref[...] = jnp.dot(h, w3_r[...], preferred_element_type=f32, precision=pr) + b3_r[...]


def _mlp(nxt, catT, g0n, b0n, g0c, b0c, W1n, W1c, b1, g1, bb1, W2, b2, g2, bb2, W3, b3):
    R = 512
    grid = (B // R,)
    col_blk = lambda shape: pl.BlockSpec(shape, lambda i: (0, i))
    full = lambda shape: pl.BlockSpec(shape, lambda i: (0, 0))
    return pl.pallas_call(
        _mlp_body,
        grid=grid,
        in_specs=[
            col_blk((NUM_DIM, R)),
            col_blk((NPLANES, R)),
            full((NUM_DIM, 1)), full((NUM_DIM, 1)),
            full((NPLANES, 1)), full((NPLANES, 1)),
            full((NUM_DIM, H1)), full((NPLANES, H1)), full((1, H1)),
            full((1, H1)), full((1, H1)),
            full((H1, H2)), full((1, H2)),
            full((1, H2)), full((1, H2)),
            full((H2, OUT)), full((1, OUT)),
        ],
        out_specs=pl.BlockSpec((R, OUT), lambda i: (i, 0)),
        out_shape=jax.ShapeDtypeStruct((B, OUT), jnp.float32),
        compiler_params=pltpu.CompilerParams(
            dimension_semantics=("arbitrary",),
        ),
    )(nxt, catT, g0n, b0n, g0c, b0c, W1n, W1c, b1, g1, bb1, W2, b2, g2, bb2, W3, b3)


def kernel(numeric_tensor, categorical_idx, tables, ln0_g, ln0_b, W1, b1,
           ln1_g, ln1_b, W2, b2, ln2_g, ln2_b, W3, b3):
    i32 = jnp.int32
    table3 = tables.transpose(0, 2, 1).reshape(RB, 8, VOCAB)
    idx26 = categorical_idx.astype(i32)

    catT = _sc_gather(table3, idx26)

    c1 = lambda a: a.reshape(-1, 1)
    r1 = lambda a: a.reshape(1, -1)
    return _mlp(
        numeric_tensor.T, catT,
        c1(ln0_g[:NUM_DIM]), c1(ln0_b[:NUM_DIM]),
        c1(ln0_g[NUM_DIM:]), c1(ln0_b[NUM_DIM:]),
        W1[:NUM_DIM], W1[NUM_DIM:], r1(b1),
        r1(ln1_g), r1(ln1_b),
        W2, r1(b2),
        r1(ln2_g), r1(ln2_b),
        W3, r1(b3),
    )
